# baseline (speedup 1.0000x reference)
import jax
import jax.numpy as jnp
from jax.experimental import pallas as pl

NUM_LAYER = 5
NUM_GRAPHS = 128


def _gin(x, edge_index, batch, W1, b1, W2, b2, eps):
    h = x
    src = edge_index[0]
    dst = edge_index[1]
    for l in range(NUM_LAYER):
        agg = jnp.zeros_like(h).at[dst].add(h[src])
        m = (1.0 + eps[l]) * h + agg
        m = jnp.maximum(m @ W1[l] + b1[l], 0.0) @ W2[l] + b2[l]
        h = jnp.maximum(m, 0.0) if l < NUM_LAYER - 1 else m
    counts = jax.ops.segment_sum(jnp.ones((h.shape[0],), dtype=h.dtype), batch, num_segments=NUM_GRAPHS)
    pooled = jax.ops.segment_sum(h, batch, num_segments=NUM_GRAPHS) / jnp.clip(counts, 1.0)[:, None]
    return pooled


def kernel(x1, x2, edge_index1, edge_index2, batch1, batch2, gin_W1, gin_b1, gin_W2, gin_b2, gin_eps, fc1_W, fc1_b, fc2_W, fc2_b, fc3_W, fc3_b):
    h1 = _gin(x1, edge_index1, batch1, gin_W1, gin_b1, gin_W2, gin_b2, gin_eps)
    h2 = _gin(x2, edge_index2, batch2, gin_W1, gin_b1, gin_W2, gin_b2, gin_eps)
    h = jnp.concatenate([h1, h2], axis=1)
    h = jnp.maximum(h @ fc1_W + fc1_b, 0.0)
    h = jnp.maximum(h @ fc2_W + fc2_b, 0.0)
    return h @ fc3_W + fc3_b


# speedup vs baseline: 2.5201x; 2.5201x over previous
import functools

import jax
import jax.numpy as jnp
from jax import lax
from jax.experimental import pallas as pl
from jax.experimental.pallas import tpu as pltpu
from jax.experimental.pallas import tpu_sc as plsc

N = 10000
E = 320000
EMB = 128
LAYERS = 5
G = 128

_NC, _NS = 2, 16
EDGES_PER_TILE = E // _NS
CHUNK = 80
NCHUNK = EDGES_PER_TILE // CHUNK
ROWS_PER_TILE = 640
TAIL_ROWS = N - 15 * ROWS_PER_TILE
N_PAD = _NS * ROWS_PER_TILE
ZROWS = 80

_sc_mesh = plsc.VectorSubcoreMesh(core_axis_name="c", subcore_axis_name="s")


@functools.partial(
    pl.kernel,
    out_type=jax.ShapeDtypeStruct((2 * N, EMB), jnp.float32),
    mesh=_sc_mesh,
    scratch_types=[
        pltpu.VMEM((CHUNK,), jnp.int32),
        pltpu.VMEM((CHUNK,), jnp.int32),
        pltpu.VMEM((CHUNK, EMB), jnp.float32),
        pltpu.VMEM((ZROWS, EMB), jnp.float32),
        pltpu.VMEM_SHARED((N_PAD, EMB), jnp.float32),
        pltpu.SemaphoreType.DMA,
    ],
)
def _sc_aggregate(h_hbm, src_hbm, dst_hbm, out_hbm, src_v, dst_v, rows_v, zbuf, agg_sh, sem):
    c = lax.axis_index("c")
    s = lax.axis_index("s")

    def _zrow(i, carry):
        for j in range(EMB // 16):
            zbuf[i, pl.ds(j * 16, 16)] = jnp.zeros((16,), jnp.float32)
        return carry

    lax.fori_loop(0, ZROWS, _zrow, 0)
    row0 = pl.multiple_of(s * ROWS_PER_TILE, 8)
    for b in range(ROWS_PER_TILE // ZROWS):
        pltpu.sync_copy(zbuf, agg_sh.at[pl.ds(row0 + b * ZROWS, ZROWS)])
    plsc.subcore_barrier()

    ebase = c * E + s * EDGES_PER_TILE

    def _body(k, carry):
        base = pl.multiple_of(ebase + k * CHUNK, 8)
        pltpu.sync_copy(src_hbm.at[pl.ds(base, CHUNK)], src_v)
        pltpu.sync_copy(dst_hbm.at[pl.ds(base, CHUNK)], dst_v)
        pltpu.async_copy(h_hbm.at[src_v], rows_v, sem).wait()
        pltpu.sync_copy(rows_v, agg_sh.at[dst_v], add=True)
        return carry

    lax.fori_loop(0, NCHUNK, _body, 0)
    plsc.subcore_barrier()

    @pl.when(s < _NS - 1)
    def _full_slice():
        pltpu.sync_copy(
            agg_sh.at[pl.ds(row0, ROWS_PER_TILE)],
            out_hbm.at[pl.ds(c * N + row0, ROWS_PER_TILE)],
        )

    @pl.when(s == _NS - 1)
    def _tail_slice():
        pltpu.sync_copy(
            agg_sh.at[pl.ds(row0, TAIL_ROWS)],
            out_hbm.at[pl.ds(c * N + row0, TAIL_ROWS)],
        )


ROWS_BLK = 2000
GRID = 2 * N // ROWS_BLK


def _mlp_mid_body(eps_ref, h_ref, agg_ref, W1_ref, b1_ref, W2_ref, b2_ref, out_ref):
    m = (1.0 + eps_ref[0, 0]) * h_ref[...] + agg_ref[...]
    t = jnp.maximum(jnp.dot(m, W1_ref[...], preferred_element_type=jnp.float32, precision=lax.Precision.HIGHEST) + b1_ref[...], 0.0)
    o = jnp.dot(t, W2_ref[...], preferred_element_type=jnp.float32, precision=lax.Precision.HIGHEST) + b2_ref[...]
    out_ref[...] = jnp.maximum(o, 0.0)


_mlp_mid = pl.pallas_call(
    _mlp_mid_body,
    grid=(GRID,),
    in_specs=[
        pl.BlockSpec(memory_space=pltpu.SMEM),
        pl.BlockSpec((ROWS_BLK, EMB), lambda i: (i, 0)),
        pl.BlockSpec((ROWS_BLK, EMB), lambda i: (i, 0)),
        pl.BlockSpec((EMB, 2 * EMB), lambda i: (0, 0)),
        pl.BlockSpec((1, 2 * EMB), lambda i: (0, 0)),
        pl.BlockSpec((2 * EMB, EMB), lambda i: (0, 0)),
        pl.BlockSpec((1, EMB), lambda i: (0, 0)),
    ],
    out_specs=pl.BlockSpec((ROWS_BLK, EMB), lambda i: (i, 0)),
    out_shape=jax.ShapeDtypeStruct((2 * N, EMB), jnp.float32),
)


def _mlp_last_body(eps_ref, h_ref, agg_ref, W1_ref, b1_ref, W2_ref, b2_ref,
                   batch_ref, pooled_ref, cnt_ref):
    m = (1.0 + eps_ref[0, 0]) * h_ref[...] + agg_ref[...]
    t = jnp.maximum(jnp.dot(m, W1_ref[...], preferred_element_type=jnp.float32, precision=lax.Precision.HIGHEST) + b1_ref[...], 0.0)
    o = jnp.dot(t, W2_ref[...], preferred_element_type=jnp.float32, precision=lax.Precision.HIGHEST) + b2_ref[...]
    seg = lax.broadcasted_iota(jnp.int32, (2 * G, ROWS_BLK), 0)
    onehotT = (seg == batch_ref[0]).astype(jnp.float32)
    pooled_c = jnp.dot(onehotT, o, preferred_element_type=jnp.float32, precision=lax.Precision.HIGHEST)
    cnt_c = jnp.dot(onehotT, jnp.ones((ROWS_BLK, EMB), jnp.float32),
                    preferred_element_type=jnp.float32, precision=lax.Precision.HIGHEST)

    @pl.when(pl.program_id(0) == 0)
    def _init():
        pooled_ref[...] = jnp.zeros_like(pooled_ref)
        cnt_ref[...] = jnp.zeros_like(cnt_ref)

    pooled_ref[...] += pooled_c
    cnt_ref[...] += cnt_c


_mlp_last = pl.pallas_call(
    _mlp_last_body,
    grid=(GRID,),
    in_specs=[
        pl.BlockSpec(memory_space=pltpu.SMEM),
        pl.BlockSpec((ROWS_BLK, EMB), lambda i: (i, 0)),
        pl.BlockSpec((ROWS_BLK, EMB), lambda i: (i, 0)),
        pl.BlockSpec((EMB, 2 * EMB), lambda i: (0, 0)),
        pl.BlockSpec((1, 2 * EMB), lambda i: (0, 0)),
        pl.BlockSpec((2 * EMB, EMB), lambda i: (0, 0)),
        pl.BlockSpec((1, EMB), lambda i: (0, 0)),
        pl.BlockSpec((1, 1, ROWS_BLK), lambda i: (i, 0, 0)),
    ],
    out_specs=[
        pl.BlockSpec((2 * G, EMB), lambda i: (0, 0)),
        pl.BlockSpec((2 * G, EMB), lambda i: (0, 0)),
    ],
    out_shape=[
        jax.ShapeDtypeStruct((2 * G, EMB), jnp.float32),
        jax.ShapeDtypeStruct((2 * G, EMB), jnp.float32),
    ],
)


def _head_body(pooled_ref, cnt_ref, W1_ref, b1_ref, W2_ref, b2_ref, W3_ref, b3_ref, out_ref):
    pooled = pooled_ref[...] / jnp.maximum(cnt_ref[...], 1.0)
    hcat = jnp.concatenate([pooled[:G, :], pooled[G:, :]], axis=1)
    t = jnp.maximum(jnp.dot(hcat, W1_ref[...], preferred_element_type=jnp.float32, precision=lax.Precision.HIGHEST) + b1_ref[...], 0.0)
    t = jnp.maximum(jnp.dot(t, W2_ref[...], preferred_element_type=jnp.float32, precision=lax.Precision.HIGHEST) + b2_ref[...], 0.0)
    out_ref[...] = jnp.dot(t, W3_ref[...], preferred_element_type=jnp.float32, precision=lax.Precision.HIGHEST) + b3_ref[...]


_head = pl.pallas_call(
    _head_body,
    out_shape=jax.ShapeDtypeStruct((G, 1), jnp.float32),
)


def kernel(x1, x2, edge_index1, edge_index2, batch1, batch2, gin_W1, gin_b1,
           gin_W2, gin_b2, gin_eps, fc1_W, fc1_b, fc2_W, fc2_b, fc3_W, fc3_b):
    h = jnp.concatenate([x1, x2], axis=0)
    src = jnp.concatenate(
        [edge_index1[0], edge_index2[0] + N], axis=0).astype(jnp.int32)
    dst = jnp.concatenate(
        [edge_index1[1], edge_index2[1]], axis=0).astype(jnp.int32)
    batch = jnp.concatenate(
        [batch1, batch2 + G], axis=0).astype(jnp.int32).reshape(GRID, 1, ROWS_BLK)

    pooled = cnt = None
    for l in range(LAYERS):
        agg = _sc_aggregate(h, src, dst)
        eps = gin_eps[l].reshape(1, 1)
        if l < LAYERS - 1:
            h = _mlp_mid(eps, h, agg, gin_W1[l], gin_b1[l].reshape(1, -1),
                         gin_W2[l], gin_b2[l].reshape(1, -1))
        else:
            pooled, cnt = _mlp_last(eps, h, agg, gin_W1[l], gin_b1[l].reshape(1, -1),
                                    gin_W2[l], gin_b2[l].reshape(1, -1), batch)
    return _head(pooled, cnt, fc1_W, fc1_b.reshape(1, -1), fc2_W,
                 fc2_b.reshape(1, -1), fc3_W, fc3_b.reshape(1, 1))
